# baseline (speedup 1.0000x reference)
import functools

import jax
import jax.numpy as jnp
from jax import lax
from jax.experimental import pallas as pl
from jax.experimental.pallas import tpu as pltpu
from jax.experimental.pallas import tpu_sc as plsc


@functools.lru_cache(maxsize=None)
def _build(B, T, D):
    info = plsc.get_sparse_core_info()
    NC, NS = info.num_cores, info.num_subcores
    NW = NC * NS
    assert B % (8 * NW) == 0
    bpw = B // NW

    mesh = plsc.VectorSubcoreMesh(core_axis_name="c", subcore_axis_name="s")

    @functools.partial(
        pl.kernel,
        mesh=mesh,
        out_type=jax.ShapeDtypeStruct((B, D), jnp.float32),
        scratch_types=[
            pltpu.VMEM((bpw,), jnp.int32),
            pltpu.VMEM((bpw,), jnp.int32),
            pltpu.VMEM((bpw, D), jnp.float32),
        ] + [pltpu.SemaphoreType.DMA] * 10,
        compiler_params=pltpu.CompilerParams(use_tc_tiling_on_sc=True),
    )
    def fused_gather(xf_hbm, len_hbm, out_hbm, len_v, idx_v, rows_v,
                     sem_l0, sem_l1, *sems):
        wid = lax.axis_index("s") * NC + lax.axis_index("c")
        base = wid * bpw
        nchunk = 4
        ck = bpw // nchunk
        lane = lax.iota(jnp.int32, 16)

        half = bpw // 2
        l0 = pltpu.make_async_copy(
            len_hbm.at[pl.ds(base, half)], len_v.at[pl.ds(0, half)], sem_l0)
        l0.start()
        l1 = pltpu.make_async_copy(
            len_hbm.at[pl.ds(base + half, half)],
            len_v.at[pl.ds(half, half)], sem_l1)
        l1.start()

        def fill_idx(lo, hi):
            for i in range(lo, hi):
                ln = len_v[pl.ds(i * 16, 16)]
                tv = jnp.where(ln == jnp.int32(0), jnp.int32(T - 1), ln - 1)
                idx_v[pl.ds(i * 16, 16)] = tv * B + (base + i * 16) + lane

        gsems = sems[:4]
        wsems = sems[4:]
        gathers = []
        l0.wait()
        for c in range(nchunk // 2):
            fill_idx(c * ck // 16, (c + 1) * ck // 16)
            g = pltpu.make_async_copy(
                xf_hbm.at[idx_v.at[pl.ds(c * ck, ck)]],
                rows_v.at[pl.ds(c * ck, ck)], gsems[c])
            g.start()
            gathers.append(g)
        l1.wait()
        for c in range(nchunk // 2, nchunk):
            fill_idx(c * ck // 16, (c + 1) * ck // 16)
            g = pltpu.make_async_copy(
                xf_hbm.at[idx_v.at[pl.ds(c * ck, ck)]],
                rows_v.at[pl.ds(c * ck, ck)], gsems[c])
            g.start()
            gathers.append(g)
        writes = []
        for c in range(nchunk):
            gathers[c].wait()
            w = pltpu.make_async_copy(
                rows_v.at[pl.ds(c * ck, ck)],
                out_hbm.at[pl.ds(base + c * ck, ck)], wsems[c])
            w.start()
            writes.append(w)
        for w in writes:
            w.wait()

    return fused_gather


def kernel(x, lengths):
    B, T, D = x.shape
    x_t = jnp.transpose(x, (1, 0, 2))
    return _build(B, T, D)(x_t.reshape(T * B, D), lengths)


# speedup vs baseline: 1.0138x; 1.0138x over previous
import functools

import jax
import jax.numpy as jnp
from jax import lax
from jax.experimental import pallas as pl
from jax.experimental.pallas import tpu as pltpu
from jax.experimental.pallas import tpu_sc as plsc


@functools.lru_cache(maxsize=None)
def _build(B, T, D):
    info = plsc.get_sparse_core_info()
    NC, NS = info.num_cores, info.num_subcores
    NW = NC * NS
    assert B % (8 * NW) == 0
    bpw = B // NW

    mesh = plsc.VectorSubcoreMesh(core_axis_name="c", subcore_axis_name="s")

    @functools.partial(
        pl.kernel,
        mesh=mesh,
        out_type=jax.ShapeDtypeStruct((B, D), jnp.float32),
        scratch_types=[
            pltpu.VMEM((bpw,), jnp.int32),
            pltpu.VMEM((bpw,), jnp.int32),
            pltpu.VMEM((bpw, D), jnp.float32),
            pltpu.SemaphoreType.DMA,
            pltpu.SemaphoreType.DMA,
            pltpu.SemaphoreType.DMA,
            pltpu.SemaphoreType.DMA,
            pltpu.SemaphoreType.DMA,
            pltpu.SemaphoreType.DMA,
        ],
        compiler_params=pltpu.CompilerParams(use_tc_tiling_on_sc=True),
    )
    def fused_gather(xf_hbm, len_hbm, out_hbm, len_v, idx_v, rows_v,
                     sem_l0, sem_l1, sem_g0, sem_g1, sem_w0, sem_w1):
        wid = lax.axis_index("s") * NC + lax.axis_index("c")
        base = wid * bpw
        half = bpw // 2
        lane = lax.iota(jnp.int32, 16)

        l0 = pltpu.make_async_copy(
            len_hbm.at[pl.ds(base, half)], len_v.at[pl.ds(0, half)], sem_l0)
        l0.start()
        l1 = pltpu.make_async_copy(
            len_hbm.at[pl.ds(base + half, half)],
            len_v.at[pl.ds(half, half)], sem_l1)
        l1.start()

        def fill_idx(lo, hi):
            for i in range(lo, hi):
                ln = len_v[pl.ds(i * 16, 16)]
                tv = jnp.where(ln == jnp.int32(0), jnp.int32(T - 1), ln - 1)
                idx_v[pl.ds(i * 16, 16)] = tv * B + (base + i * 16) + lane

        l0.wait()
        fill_idx(0, half // 16)
        g0 = pltpu.make_async_copy(
            xf_hbm.at[idx_v.at[pl.ds(0, half)]], rows_v.at[pl.ds(0, half)],
            sem_g0)
        g0.start()
        l1.wait()
        fill_idx(half // 16, bpw // 16)
        g1 = pltpu.make_async_copy(
            xf_hbm.at[idx_v.at[pl.ds(half, half)]],
            rows_v.at[pl.ds(half, half)], sem_g1)
        g1.start()
        g0.wait()
        w0 = pltpu.make_async_copy(
            rows_v.at[pl.ds(0, half)], out_hbm.at[pl.ds(base, half)], sem_w0)
        w0.start()
        g1.wait()
        w1 = pltpu.make_async_copy(
            rows_v.at[pl.ds(half, half)],
            out_hbm.at[pl.ds(base + half, half)], sem_w1)
        w1.start()
        w0.wait()
        w1.wait()

    return fused_gather


def kernel(x, lengths):
    B, T, D = x.shape
    x_t = jnp.transpose(x, (1, 0, 2))
    return _build(B, T, D)(x_t.reshape(T * B, D), lengths)
